# baseline (speedup 1.0000x reference)
import functools

import jax
import jax.numpy as jnp
from jax import lax
from jax.experimental import pallas as pl
from jax.experimental.pallas import tpu as pltpu
from jax.experimental.pallas import tpu_sc as plsc


def _argmax_body(vocab, x_ref, out_ref):
    blk = x_ref[...]
    m = jnp.max(blk, axis=-1, keepdims=True)
    pos = lax.broadcasted_iota(jnp.int32, blk.shape, 2)
    idx = jnp.min(jnp.where(blk == m, pos, vocab), axis=-1)
    out_ref[...] = idx


def _tc_argmax(x, bb, b0=0, nb=None):
    b, n, v = x.shape
    if nb is None:
        nb = b // bb
    blk0 = b0 // bb
    return pl.pallas_call(
        functools.partial(_argmax_body, v),
        grid=(nb,),
        in_specs=[pl.BlockSpec((bb, n, v), lambda i: (blk0 + i, 0, 0))],
        out_specs=pl.BlockSpec((bb, n), lambda i: (i, 0)),
        out_shape=jax.ShapeDtypeStruct((nb * bb, n), jnp.int32),
    )(x)


def _sc_argmax(x):
    b, n, v = x.shape
    info = plsc.get_sparse_core_info()
    nc, ns = info.num_cores, info.num_subcores
    nw = nc * ns
    spw = b // nw
    mesh = plsc.VectorSubcoreMesh(core_axis_name="c", subcore_axis_name="s")

    bases = list(range(0, v - 15, 16))
    if bases[-1] + 16 < v:
        bases.append(v - 16)
    neg_inf = jnp.float32(-3.402823466e38)

    nt = 64

    @functools.partial(
        pl.kernel,
        mesh=mesh,
        out_type=jax.ShapeDtypeStruct((b, nt), jnp.int32),
        scratch_types=[
            pltpu.VMEM((2, n, v), jnp.float32),
            pltpu.VMEM((spw, nt), jnp.int32),
            pltpu.SemaphoreType.DMA,
            pltpu.SemaphoreType.DMA,
        ],
        compiler_params=pltpu.CompilerParams(
            needs_layout_passes=False, use_tc_tiling_on_sc=True
        ),
    )
    def _k(x_hbm, tok_hbm, xs_v, tok_v, sem0, sem1):
        wid = lax.axis_index("s") * nc + lax.axis_index("c")
        b0 = wid * spw
        lanes = lax.iota(jnp.int32, 16)

        n_acc = 4

        def compute_slab(s, buf):
            def n_body(nn, acc_tok):
                acc_v = [jnp.full((16,), neg_inf, jnp.float32)] * n_acc
                acc_j = [jnp.zeros((16,), jnp.int32)] * n_acc
                for i, base in enumerate(bases):
                    k = i % n_acc
                    val = buf[nn, pl.ds(base, 16)]
                    gt = val > acc_v[k]
                    acc_v[k] = jnp.maximum(val, acc_v[k])
                    acc_j[k] = jnp.where(
                        gt, jnp.full((16,), base, jnp.int32), acc_j[k]
                    )
                m = jnp.max(jnp.maximum(
                    jnp.maximum(acc_v[0], acc_v[1]),
                    jnp.maximum(acc_v[2], acc_v[3]),
                ))
                cand = jnp.full((16,), v, jnp.int32)
                for k in range(n_acc):
                    cand = jnp.minimum(
                        cand, jnp.where(acc_v[k] == m, acc_j[k] + lanes, v)
                    )
                tok = jnp.min(cand)
                acc_tok = jnp.where(lanes == nn % 16, tok, acc_tok)

                @pl.when((nn % 16 == 15) | (nn == n - 1))
                def _():
                    tok_v[s, pl.ds((nn // 16) * 16, 16)] = acc_tok

                return acc_tok

            lax.fori_loop(0, n, n_body, jnp.zeros((16,), jnp.int32))

        pltpu.make_async_copy(x_hbm.at[b0], xs_v.at[0], sem0).start()

        def g_body(g, carry):
            s0 = 2 * g

            @pl.when(s0 + 1 < spw)
            def _():
                pltpu.make_async_copy(
                    x_hbm.at[b0 + s0 + 1], xs_v.at[1], sem1
                ).start()

            pltpu.make_async_copy(x_hbm.at[b0], xs_v.at[0], sem0).wait()
            compute_slab(s0, xs_v.at[0])

            @pl.when(s0 + 2 < spw)
            def _():
                pltpu.make_async_copy(
                    x_hbm.at[b0 + s0 + 2], xs_v.at[0], sem0
                ).start()

            pltpu.make_async_copy(x_hbm.at[b0], xs_v.at[1], sem1).wait()
            compute_slab(s0 + 1, xs_v.at[1])
            return carry

        lax.fori_loop(0, spw // 2, g_body, 0)
        pltpu.sync_copy(tok_v, tok_hbm.at[pl.ds(b0, spw)])

    return _k(x)


def _sc_gather(table, tokens, n_chunks=1, untiled=False):
    m = tokens.shape[0]
    d = table.shape[1]
    info = plsc.get_sparse_core_info()
    nc, ns = info.num_cores, info.num_subcores
    nw = nc * ns
    b_per_w = m // nw
    bc = b_per_w // n_chunks
    mesh = plsc.VectorSubcoreMesh(core_axis_name="c", subcore_axis_name="s")
    params = pltpu.CompilerParams(use_tc_tiling_on_sc=False) if untiled else None

    @functools.partial(
        pl.kernel,
        mesh=mesh,
        out_type=jax.ShapeDtypeStruct((m, d), jnp.float32),
        scratch_types=[
            pltpu.VMEM((b_per_w,), jnp.int32),
            pltpu.VMEM((bc, d), jnp.float32),
            pltpu.SemaphoreType.DMA,
        ],
        compiler_params=params,
    )
    def _gather(table_hbm, idx_hbm, out_hbm, idx_v, rows_v, sem):
        wid = lax.axis_index("s") * nc + lax.axis_index("c")
        base = wid * b_per_w
        pltpu.sync_copy(idx_hbm.at[pl.ds(base, b_per_w)], idx_v)
        for c in range(n_chunks):
            src = idx_v if n_chunks == 1 else idx_v.at[pl.ds(c * bc, bc)]
            pltpu.async_copy(table_hbm.at[src], rows_v, sem).wait()
            pltpu.sync_copy(rows_v, out_hbm.at[pl.ds(base + c * bc, bc)])

    return _gather(table, tokens)


def kernel(x, W):
    b, n, v = x.shape
    d = W.shape[1]
    tokens = _sc_argmax(x)[:, :n].reshape(b * n)
    out = _sc_gather(W, tokens, n_chunks=1, untiled=True)
    return out.reshape(b, n, d)


# speedup vs baseline: 1.3172x; 1.3172x over previous
import functools

import jax
import jax.numpy as jnp
from jax import lax
from jax.experimental import pallas as pl
from jax.experimental.pallas import tpu as pltpu
from jax.experimental.pallas import tpu_sc as plsc


def _argmax_body(vocab, x_ref, out_ref):
    blk = x_ref[...]
    m = jnp.max(blk, axis=-1, keepdims=True)
    pos = lax.broadcasted_iota(jnp.int32, blk.shape, 2)
    idx = jnp.min(jnp.where(blk == m, pos, vocab), axis=-1)
    out_ref[...] = idx


def _tc_argmax(x, bb, b0=0, nb=None):
    b, n, v = x.shape
    if nb is None:
        nb = b // bb
    blk0 = b0 // bb
    return pl.pallas_call(
        functools.partial(_argmax_body, v),
        grid=(nb,),
        in_specs=[pl.BlockSpec((bb, n, v), lambda i: (blk0 + i, 0, 0))],
        out_specs=pl.BlockSpec((bb, n), lambda i: (i, 0)),
        out_shape=jax.ShapeDtypeStruct((nb * bb, n), jnp.int32),
    )(x)


def _sc_argmax(x):
    b, n, v = x.shape
    info = plsc.get_sparse_core_info()
    nc, ns = info.num_cores, info.num_subcores
    nw = nc * ns
    spw = b // nw
    mesh = plsc.VectorSubcoreMesh(core_axis_name="c", subcore_axis_name="s")

    bases = list(range(0, v - 15, 16))
    if bases[-1] + 16 < v:
        bases.append(v - 16)
    neg_inf = jnp.float32(-3.402823466e38)

    nt = 64

    @functools.partial(
        pl.kernel,
        mesh=mesh,
        out_type=jax.ShapeDtypeStruct((b, nt), jnp.int32),
        scratch_types=[
            pltpu.VMEM((2, n, v), jnp.float32),
            pltpu.VMEM((spw, nt), jnp.int32),
            pltpu.SemaphoreType.DMA,
            pltpu.SemaphoreType.DMA,
        ],
        compiler_params=pltpu.CompilerParams(
            needs_layout_passes=False, use_tc_tiling_on_sc=True
        ),
    )
    def _k(x_hbm, tok_hbm, xs_v, tok_v, sem0, sem1):
        wid = lax.axis_index("s") * nc + lax.axis_index("c")
        b0 = wid * spw
        lanes = lax.iota(jnp.int32, 16)

        n_acc = 4

        def compute_slab(s, buf):
            def n_body(nn, acc_tok):
                acc_v = [jnp.full((16,), neg_inf, jnp.float32)] * n_acc
                acc_j = [jnp.zeros((16,), jnp.int32)] * n_acc
                for i, base in enumerate(bases):
                    k = i % n_acc
                    val = buf[nn, pl.ds(base, 16)]
                    gt = val > acc_v[k]
                    acc_v[k] = jnp.maximum(val, acc_v[k])
                    acc_j[k] = jnp.where(
                        gt, jnp.full((16,), base, jnp.int32), acc_j[k]
                    )
                m = jnp.max(jnp.maximum(
                    jnp.maximum(acc_v[0], acc_v[1]),
                    jnp.maximum(acc_v[2], acc_v[3]),
                ))
                cand = jnp.full((16,), v, jnp.int32)
                for k in range(n_acc):
                    cand = jnp.minimum(
                        cand, jnp.where(acc_v[k] == m, acc_j[k] + lanes, v)
                    )
                tok = jnp.min(cand)
                acc_tok = jnp.where(lanes == nn % 16, tok, acc_tok)

                @pl.when((nn % 16 == 15) | (nn == n - 1))
                def _():
                    tok_v[s, pl.ds((nn // 16) * 16, 16)] = acc_tok

                return acc_tok

            lax.fori_loop(0, n, n_body, jnp.zeros((16,), jnp.int32))

        pltpu.make_async_copy(x_hbm.at[b0], xs_v.at[0], sem0).start()

        def g_body(g, carry):
            s0 = 2 * g

            @pl.when(s0 + 1 < spw)
            def _():
                pltpu.make_async_copy(
                    x_hbm.at[b0 + s0 + 1], xs_v.at[1], sem1
                ).start()

            pltpu.make_async_copy(x_hbm.at[b0], xs_v.at[0], sem0).wait()
            compute_slab(s0, xs_v.at[0])

            @pl.when(s0 + 2 < spw)
            def _():
                pltpu.make_async_copy(
                    x_hbm.at[b0 + s0 + 2], xs_v.at[0], sem0
                ).start()

            pltpu.make_async_copy(x_hbm.at[b0], xs_v.at[1], sem1).wait()
            compute_slab(s0 + 1, xs_v.at[1])
            return carry

        lax.fori_loop(0, spw // 2, g_body, 0)
        pltpu.sync_copy(tok_v, tok_hbm.at[pl.ds(b0, spw)])

    return _k(x)


def _sc_gather(table, tokens, n_chunks=1, untiled=False):
    m = tokens.shape[0]
    d = table.shape[1]
    info = plsc.get_sparse_core_info()
    nc, ns = info.num_cores, info.num_subcores
    nw = nc * ns
    b_per_w = m // nw
    bc = b_per_w // n_chunks
    mesh = plsc.VectorSubcoreMesh(core_axis_name="c", subcore_axis_name="s")
    params = pltpu.CompilerParams(use_tc_tiling_on_sc=False) if untiled else None

    @functools.partial(
        pl.kernel,
        mesh=mesh,
        out_type=jax.ShapeDtypeStruct((m, d), jnp.float32),
        scratch_types=[
            pltpu.VMEM((b_per_w,), jnp.int32),
            pltpu.VMEM((bc, d), jnp.float32),
            pltpu.SemaphoreType.DMA,
        ],
        compiler_params=params,
    )
    def _gather(table_hbm, idx_hbm, out_hbm, idx_v, rows_v, sem):
        wid = lax.axis_index("s") * nc + lax.axis_index("c")
        base = wid * b_per_w
        pltpu.sync_copy(idx_hbm.at[pl.ds(base, b_per_w)], idx_v)
        for c in range(n_chunks):
            src = idx_v if n_chunks == 1 else idx_v.at[pl.ds(c * bc, bc)]
            pltpu.async_copy(table_hbm.at[src], rows_v, sem).wait()
            pltpu.sync_copy(rows_v, out_hbm.at[pl.ds(base + c * bc, bc)])

    return _gather(table, tokens)


def kernel(x, W):
    b, n, v = x.shape
    d = W.shape[1]
    tokens = _tc_argmax(x, bb=32).reshape(b * n)
    out = _sc_gather(W, tokens, n_chunks=1, untiled=True)
    return out.reshape(b, n, d)


# speedup vs baseline: 1.3385x; 1.0162x over previous
import functools

import jax
import jax.numpy as jnp
from jax import lax
from jax.experimental import pallas as pl
from jax.experimental.pallas import tpu as pltpu
from jax.experimental.pallas import tpu_sc as plsc


def _argmax_body(vocab, x_ref, out_ref):
    blk = x_ref[...]
    m = jnp.max(blk, axis=-1, keepdims=True)
    pos = lax.broadcasted_iota(jnp.int32, blk.shape, 2)
    idx = jnp.min(jnp.where(blk == m, pos, vocab), axis=-1)
    out_ref[...] = idx


def _tc_argmax(x, bb, b0=0, nb=None):
    b, n, v = x.shape
    if nb is None:
        nb = b // bb
    blk0 = b0 // bb
    return pl.pallas_call(
        functools.partial(_argmax_body, v),
        grid=(nb,),
        in_specs=[pl.BlockSpec((bb, n, v), lambda i: (blk0 + i, 0, 0))],
        out_specs=pl.BlockSpec((bb, n), lambda i: (i, 0)),
        out_shape=jax.ShapeDtypeStruct((nb * bb, n), jnp.int32),
    )(x)


def _sc_argmax(x):
    b, n, v = x.shape
    info = plsc.get_sparse_core_info()
    nc, ns = info.num_cores, info.num_subcores
    nw = nc * ns
    spw = b // nw
    mesh = plsc.VectorSubcoreMesh(core_axis_name="c", subcore_axis_name="s")

    bases = list(range(0, v - 15, 16))
    if bases[-1] + 16 < v:
        bases.append(v - 16)
    neg_inf = jnp.float32(-3.402823466e38)

    nt = 64

    @functools.partial(
        pl.kernel,
        mesh=mesh,
        out_type=jax.ShapeDtypeStruct((b, nt), jnp.int32),
        scratch_types=[
            pltpu.VMEM((2, n, v), jnp.float32),
            pltpu.VMEM((spw, nt), jnp.int32),
            pltpu.SemaphoreType.DMA,
            pltpu.SemaphoreType.DMA,
        ],
        compiler_params=pltpu.CompilerParams(
            needs_layout_passes=False, use_tc_tiling_on_sc=True
        ),
    )
    def _k(x_hbm, tok_hbm, xs_v, tok_v, sem0, sem1):
        wid = lax.axis_index("s") * nc + lax.axis_index("c")
        b0 = wid * spw
        lanes = lax.iota(jnp.int32, 16)

        n_acc = 4

        def compute_slab(s, buf):
            def n_body(nn, acc_tok):
                acc_v = [jnp.full((16,), neg_inf, jnp.float32)] * n_acc
                acc_j = [jnp.zeros((16,), jnp.int32)] * n_acc
                for i, base in enumerate(bases):
                    k = i % n_acc
                    val = buf[nn, pl.ds(base, 16)]
                    gt = val > acc_v[k]
                    acc_v[k] = jnp.maximum(val, acc_v[k])
                    acc_j[k] = jnp.where(
                        gt, jnp.full((16,), base, jnp.int32), acc_j[k]
                    )
                m = jnp.max(jnp.maximum(
                    jnp.maximum(acc_v[0], acc_v[1]),
                    jnp.maximum(acc_v[2], acc_v[3]),
                ))
                cand = jnp.full((16,), v, jnp.int32)
                for k in range(n_acc):
                    cand = jnp.minimum(
                        cand, jnp.where(acc_v[k] == m, acc_j[k] + lanes, v)
                    )
                tok = jnp.min(cand)
                acc_tok = jnp.where(lanes == nn % 16, tok, acc_tok)

                @pl.when((nn % 16 == 15) | (nn == n - 1))
                def _():
                    tok_v[s, pl.ds((nn // 16) * 16, 16)] = acc_tok

                return acc_tok

            lax.fori_loop(0, n, n_body, jnp.zeros((16,), jnp.int32))

        pltpu.make_async_copy(x_hbm.at[b0], xs_v.at[0], sem0).start()

        def g_body(g, carry):
            s0 = 2 * g

            @pl.when(s0 + 1 < spw)
            def _():
                pltpu.make_async_copy(
                    x_hbm.at[b0 + s0 + 1], xs_v.at[1], sem1
                ).start()

            pltpu.make_async_copy(x_hbm.at[b0], xs_v.at[0], sem0).wait()
            compute_slab(s0, xs_v.at[0])

            @pl.when(s0 + 2 < spw)
            def _():
                pltpu.make_async_copy(
                    x_hbm.at[b0 + s0 + 2], xs_v.at[0], sem0
                ).start()

            pltpu.make_async_copy(x_hbm.at[b0], xs_v.at[1], sem1).wait()
            compute_slab(s0 + 1, xs_v.at[1])
            return carry

        lax.fori_loop(0, spw // 2, g_body, 0)
        pltpu.sync_copy(tok_v, tok_hbm.at[pl.ds(b0, spw)])

    return _k(x)


def _sc_gather(table, tokens, n_chunks=1, untiled=False):
    m = tokens.shape[0]
    d = table.shape[1]
    info = plsc.get_sparse_core_info()
    nc, ns = info.num_cores, info.num_subcores
    nw = nc * ns
    b_per_w = m // nw
    bc = b_per_w // n_chunks
    mesh = plsc.VectorSubcoreMesh(core_axis_name="c", subcore_axis_name="s")
    params = pltpu.CompilerParams(use_tc_tiling_on_sc=False) if untiled else None

    @functools.partial(
        pl.kernel,
        mesh=mesh,
        out_type=jax.ShapeDtypeStruct((m, d), jnp.float32),
        scratch_types=[
            pltpu.VMEM((b_per_w,), jnp.int32),
            pltpu.VMEM((bc, d), jnp.float32),
            pltpu.SemaphoreType.DMA,
        ],
        compiler_params=params,
    )
    def _gather(table_hbm, idx_hbm, out_hbm, idx_v, rows_v, sem):
        wid = lax.axis_index("s") * nc + lax.axis_index("c")
        base = wid * b_per_w
        pltpu.sync_copy(idx_hbm.at[pl.ds(base, b_per_w)], idx_v)
        for c in range(n_chunks):
            src = idx_v if n_chunks == 1 else idx_v.at[pl.ds(c * bc, bc)]
            pltpu.async_copy(table_hbm.at[src], rows_v, sem).wait()
            pltpu.sync_copy(rows_v, out_hbm.at[pl.ds(base + c * bc, bc)])

    return _gather(table, tokens)


def kernel(x, W):
    b, n, v = x.shape
    d = W.shape[1]
    tokens = _tc_argmax(x, bb=64).reshape(b * n)
    out = _sc_gather(W, tokens, n_chunks=1, untiled=True)
    return out.reshape(b, n, d)


# speedup vs baseline: 1.3431x; 1.0034x over previous
import functools

import jax
import jax.numpy as jnp
from jax import lax
from jax.experimental import pallas as pl
from jax.experimental.pallas import tpu as pltpu
from jax.experimental.pallas import tpu_sc as plsc


def _argmax_body(vocab, x_ref, out_ref):
    blk = x_ref[...]
    m = jnp.max(blk, axis=-1, keepdims=True)
    pos = lax.broadcasted_iota(jnp.int32, blk.shape, 2)
    idx = jnp.min(jnp.where(blk == m, pos, vocab), axis=-1)
    out_ref[...] = idx


def _tc_argmax(x, bb, b0=0, nb=None):
    b, n, v = x.shape
    if nb is None:
        nb = b // bb
    blk0 = b0 // bb
    return pl.pallas_call(
        functools.partial(_argmax_body, v),
        grid=(nb,),
        in_specs=[pl.BlockSpec((bb, n, v), lambda i: (blk0 + i, 0, 0))],
        out_specs=pl.BlockSpec((bb, n), lambda i: (i, 0)),
        out_shape=jax.ShapeDtypeStruct((nb * bb, n), jnp.int32),
        compiler_params=pltpu.CompilerParams(needs_layout_passes=False),
    )(x)


def _sc_argmax(x):
    b, n, v = x.shape
    info = plsc.get_sparse_core_info()
    nc, ns = info.num_cores, info.num_subcores
    nw = nc * ns
    spw = b // nw
    mesh = plsc.VectorSubcoreMesh(core_axis_name="c", subcore_axis_name="s")

    bases = list(range(0, v - 15, 16))
    if bases[-1] + 16 < v:
        bases.append(v - 16)
    neg_inf = jnp.float32(-3.402823466e38)

    nt = 64

    @functools.partial(
        pl.kernel,
        mesh=mesh,
        out_type=jax.ShapeDtypeStruct((b, nt), jnp.int32),
        scratch_types=[
            pltpu.VMEM((2, n, v), jnp.float32),
            pltpu.VMEM((spw, nt), jnp.int32),
            pltpu.SemaphoreType.DMA,
            pltpu.SemaphoreType.DMA,
        ],
        compiler_params=pltpu.CompilerParams(
            needs_layout_passes=False, use_tc_tiling_on_sc=True
        ),
    )
    def _k(x_hbm, tok_hbm, xs_v, tok_v, sem0, sem1):
        wid = lax.axis_index("s") * nc + lax.axis_index("c")
        b0 = wid * spw
        lanes = lax.iota(jnp.int32, 16)

        n_acc = 4

        def compute_slab(s, buf):
            def n_body(nn, acc_tok):
                acc_v = [jnp.full((16,), neg_inf, jnp.float32)] * n_acc
                acc_j = [jnp.zeros((16,), jnp.int32)] * n_acc
                for i, base in enumerate(bases):
                    k = i % n_acc
                    val = buf[nn, pl.ds(base, 16)]
                    gt = val > acc_v[k]
                    acc_v[k] = jnp.maximum(val, acc_v[k])
                    acc_j[k] = jnp.where(
                        gt, jnp.full((16,), base, jnp.int32), acc_j[k]
                    )
                m = jnp.max(jnp.maximum(
                    jnp.maximum(acc_v[0], acc_v[1]),
                    jnp.maximum(acc_v[2], acc_v[3]),
                ))
                cand = jnp.full((16,), v, jnp.int32)
                for k in range(n_acc):
                    cand = jnp.minimum(
                        cand, jnp.where(acc_v[k] == m, acc_j[k] + lanes, v)
                    )
                tok = jnp.min(cand)
                acc_tok = jnp.where(lanes == nn % 16, tok, acc_tok)

                @pl.when((nn % 16 == 15) | (nn == n - 1))
                def _():
                    tok_v[s, pl.ds((nn // 16) * 16, 16)] = acc_tok

                return acc_tok

            lax.fori_loop(0, n, n_body, jnp.zeros((16,), jnp.int32))

        pltpu.make_async_copy(x_hbm.at[b0], xs_v.at[0], sem0).start()

        def g_body(g, carry):
            s0 = 2 * g

            @pl.when(s0 + 1 < spw)
            def _():
                pltpu.make_async_copy(
                    x_hbm.at[b0 + s0 + 1], xs_v.at[1], sem1
                ).start()

            pltpu.make_async_copy(x_hbm.at[b0], xs_v.at[0], sem0).wait()
            compute_slab(s0, xs_v.at[0])

            @pl.when(s0 + 2 < spw)
            def _():
                pltpu.make_async_copy(
                    x_hbm.at[b0 + s0 + 2], xs_v.at[0], sem0
                ).start()

            pltpu.make_async_copy(x_hbm.at[b0], xs_v.at[1], sem1).wait()
            compute_slab(s0 + 1, xs_v.at[1])
            return carry

        lax.fori_loop(0, spw // 2, g_body, 0)
        pltpu.sync_copy(tok_v, tok_hbm.at[pl.ds(b0, spw)])

    return _k(x)


def _sc_gather(table, tokens, n_chunks=1, untiled=False):
    m = tokens.shape[0]
    d = table.shape[1]
    info = plsc.get_sparse_core_info()
    nc, ns = info.num_cores, info.num_subcores
    nw = nc * ns
    b_per_w = m // nw
    bc = b_per_w // n_chunks
    mesh = plsc.VectorSubcoreMesh(core_axis_name="c", subcore_axis_name="s")
    params = pltpu.CompilerParams(use_tc_tiling_on_sc=False) if untiled else None

    @functools.partial(
        pl.kernel,
        mesh=mesh,
        out_type=jax.ShapeDtypeStruct((m, d), jnp.float32),
        scratch_types=[
            pltpu.VMEM((b_per_w,), jnp.int32),
            pltpu.VMEM((bc, d), jnp.float32),
            pltpu.SemaphoreType.DMA,
        ],
        compiler_params=params,
    )
    def _gather(table_hbm, idx_hbm, out_hbm, idx_v, rows_v, sem):
        wid = lax.axis_index("s") * nc + lax.axis_index("c")
        base = wid * b_per_w
        pltpu.sync_copy(idx_hbm.at[pl.ds(base, b_per_w)], idx_v)
        for c in range(n_chunks):
            src = idx_v if n_chunks == 1 else idx_v.at[pl.ds(c * bc, bc)]
            pltpu.async_copy(table_hbm.at[src], rows_v, sem).wait()
            pltpu.sync_copy(rows_v, out_hbm.at[pl.ds(base + c * bc, bc)])

    return _gather(table, tokens)


def kernel(x, W):
    b, n, v = x.shape
    d = W.shape[1]
    tokens = _tc_argmax(x, bb=64).reshape(b * n)
    out = _sc_gather(W, tokens, n_chunks=1, untiled=True)
    return out.reshape(b, n, d)


# speedup vs baseline: 1.3465x; 1.0025x over previous
import functools

import jax
import jax.numpy as jnp
from jax import lax
from jax.experimental import pallas as pl
from jax.experimental.pallas import tpu as pltpu
from jax.experimental.pallas import tpu_sc as plsc


def _argmax_body(vocab, x_ref, out_ref):
    blk = x_ref[...]
    m = jnp.max(blk, axis=-1, keepdims=True)
    pos = lax.broadcasted_iota(jnp.int32, blk.shape, 2)
    idx = jnp.min(jnp.where(blk == m, pos, vocab), axis=-1)
    out_ref[...] = idx


def _tc_argmax(x, bb):
    b, n, v = x.shape
    nb = b // bb
    return pl.pallas_call(
        functools.partial(_argmax_body, v),
        grid=(nb,),
        in_specs=[pl.BlockSpec((bb, n, v), lambda i: (i, 0, 0))],
        out_specs=pl.BlockSpec((bb, n), lambda i: (i, 0)),
        out_shape=jax.ShapeDtypeStruct((b, n), jnp.int32),
    )(x)


def _sc_gather(table, tokens):
    m = tokens.shape[0]
    d = table.shape[1]
    info = plsc.get_sparse_core_info()
    nc, ns = info.num_cores, info.num_subcores
    nw = nc * ns
    b_per_w = m // nw
    mesh = plsc.VectorSubcoreMesh(core_axis_name="c", subcore_axis_name="s")

    @functools.partial(
        pl.kernel,
        mesh=mesh,
        out_type=jax.ShapeDtypeStruct((m, d), jnp.float32),
        scratch_types=[
            pltpu.VMEM((b_per_w,), jnp.int32),
            pltpu.VMEM((b_per_w, d), jnp.float32),
            pltpu.SemaphoreType.DMA,
        ],
        compiler_params=pltpu.CompilerParams(use_tc_tiling_on_sc=False),
    )
    def _gather(table_hbm, idx_hbm, out_hbm, idx_v, rows_v, sem):
        wid = lax.axis_index("s") * nc + lax.axis_index("c")
        base = wid * b_per_w
        pltpu.sync_copy(idx_hbm.at[pl.ds(base, b_per_w)], idx_v)
        pltpu.async_copy(table_hbm.at[idx_v], rows_v, sem).wait()
        pltpu.sync_copy(rows_v, out_hbm.at[pl.ds(base, b_per_w)])

    return _gather(table, tokens)


def kernel(x, W):
    b, n, v = x.shape
    d = W.shape[1]
    tokens = _tc_argmax(x, bb=64).reshape(b * n)
    out = _sc_gather(W, tokens)
    return out.reshape(b, n, d)
